# baseline (speedup 1.0000x reference)
import functools

import jax
import jax.numpy as jnp
from jax import lax
from jax.experimental import pallas as pl
from jax.experimental.pallas import tpu as pltpu
from jax.experimental.pallas import tpu_sc as plsc

VOCAB = 1000000
EMBED_DIM = 64
BATCH = 16384

NUM_CORES = 2
NUM_SUBCORES = 16
LANES = 16
NUM_WORKERS = NUM_CORES * NUM_SUBCORES
BPW = BATCH // NUM_WORKERS
CHUNK = 128
HALF = BPW // 2


def _word2vec_body(tgt_hbm, ctx_hbm, tin_hbm, tout_hbm, out_hbm,
                   jt_v, jc_v, ot_v, oc_v, trows, crows, out_v,
                   sem_t, sem_c):
    wid = lax.axis_index("s") * NUM_CORES + lax.axis_index("c")
    base = wid * BPW

    pltpu.sync_copy(tgt_hbm.at[pl.ds(base, BPW)], ot_v)
    pltpu.sync_copy(ctx_hbm.at[pl.ds(base, BPW)], oc_v)

    def split_block(i, carry):
        t = ot_v[pl.ds(i * LANES, LANES)]
        c = oc_v[pl.ds(i * LANES, LANES)]
        jrow = i // (CHUNK // LANES)
        jcol = (i % (CHUNK // LANES)) * LANES
        jt_v[jrow, pl.ds(jcol, LANES)] = lax.shift_right_logical(t, 1)
        jc_v[jrow, pl.ds(jcol, LANES)] = lax.shift_right_logical(c, 1)
        ot_v[pl.ds(i * LANES, LANES)] = (t & 1) * EMBED_DIM
        oc_v[pl.ds(i * LANES, LANES)] = (c & 1) * EMBED_DIM
        return carry

    lax.fori_loop(0, BPW // LANES, split_block, 0)

    lanes = lax.iota(jnp.int32, LANES)

    for half in range(2):
        h0 = half * HALF
        copies = []
        for j in range(HALF // CHUNK):
            jrow = (h0 + j * CHUNK) // CHUNK
            copies.append(pltpu.async_copy(
                tin_hbm.at[jt_v.at[jrow]],
                trows.at[pl.ds(j * CHUNK, CHUNK)], sem_t))
            copies.append(pltpu.async_copy(
                tout_hbm.at[jc_v.at[jrow]],
                crows.at[pl.ds(j * CHUNK, CHUNK)], sem_c))
        for cp in copies:
            cp.wait()

        def block(i, carry):
            r0 = i * LANES
            offt = ot_v[pl.ds(h0 + r0, LANES)]
            offc = oc_v[pl.ds(h0 + r0, LANES)]
            acc = jnp.zeros((LANES,), jnp.float32)
            for r in range(LANES):
                ot = offt[r]
                oc = offc[r]
                s = jnp.zeros((LANES,), jnp.float32)
                for k in range(EMBED_DIM // LANES):
                    tv = trows[r0 + r, pl.ds(ot + k * LANES, LANES)]
                    cv = crows[r0 + r, pl.ds(oc + k * LANES, LANES)]
                    s = s + tv * cv
                tot = jnp.sum(s)
                acc = jnp.where(lanes == r, tot, acc)
            out_v[pl.ds(h0 + r0, LANES)] = acc
            return carry

        lax.fori_loop(0, HALF // LANES, block, 0)

    pltpu.sync_copy(out_v, out_hbm.at[pl.ds(base, BPW)])


@jax.jit
def _word2vec(target, context, tin_p, tout_p):
    mesh = plsc.VectorSubcoreMesh(core_axis_name="c", subcore_axis_name="s")
    return pl.kernel(
        _word2vec_body,
        mesh=mesh,
        compiler_params=pltpu.CompilerParams(
            needs_layout_passes=False, use_tc_tiling_on_sc=True),
        out_type=jax.ShapeDtypeStruct((BATCH,), jnp.float32),
        scratch_types=[
            pltpu.VMEM((BPW // CHUNK, CHUNK), jnp.int32),
            pltpu.VMEM((BPW // CHUNK, CHUNK), jnp.int32),
            pltpu.VMEM((BPW,), jnp.int32),
            pltpu.VMEM((BPW,), jnp.int32),
            pltpu.VMEM((HALF, 2 * EMBED_DIM), jnp.float32),
            pltpu.VMEM((HALF, 2 * EMBED_DIM), jnp.float32),
            pltpu.VMEM((BPW,), jnp.float32),
            pltpu.SemaphoreType.DMA,
            pltpu.SemaphoreType.DMA,
        ],
    )(target, context, tin_p, tout_p)


def kernel(target, context, in_table, out_table):
    tin_p = in_table.reshape(VOCAB // 2, 2 * EMBED_DIM)
    tout_p = out_table.reshape(VOCAB // 2, 2 * EMBED_DIM)
    return _word2vec(target.astype(jnp.int32), context.astype(jnp.int32),
                     tin_p, tout_p)


# speedup vs baseline: 1.2657x; 1.2657x over previous
import functools

import jax
import jax.numpy as jnp
from jax import lax
from jax.experimental import pallas as pl
from jax.experimental.pallas import tpu as pltpu
from jax.experimental.pallas import tpu_sc as plsc

VOCAB = 1000000
EMBED_DIM = 64
BATCH = 16384

NUM_CORES = 2
NUM_SUBCORES = 16
LANES = 16
NUM_WORKERS = NUM_CORES * NUM_SUBCORES
NCOL = (VOCAB + 127) // 128
SHARD_COLS = 496
BOUNDS_PAD = 15 * SHARD_COLS + 512
EMB_ROWS = BATCH + LANES
BPW = BATCH // NUM_WORKERS
QUARTER = 128

_COMPILER_PARAMS = pltpu.CompilerParams(
    needs_layout_passes=False, use_tc_tiling_on_sc=True,
    disable_bounds_checks=True)


def _sload(ref, i):
    return ref[pl.ds(i, LANES)][0]


def _gather_body(tin_hbm, tout_hbm, sidx_t_hbm, perm_t_hbm, bnd_t_hbm,
                 sidx_c_hbm, perm_c_hbm, bnd_c_hbm, embt_hbm, embc_hbm,
                 sidx_v, perm_v, bnd_v, colbuf0, colbuf1, outstage, posbuf,
                 sem0, sem1, semsc):
    cid = lax.axis_index("c")
    shard = lax.axis_index("s")
    col0 = shard * SHARD_COLS
    lanes = lax.iota(jnp.int32, LANES)
    dump_init = BATCH + lanes

    def process(tbl_hbm, sidx_hbm, perm_hbm, bnd_hbm, emb_hbm):
        pltpu.sync_copy(sidx_hbm, sidx_v)
        pltpu.sync_copy(perm_hbm, perm_v)
        pltpu.sync_copy(bnd_hbm.at[pl.ds(col0, SHARD_COLS + 16)], bnd_v)

        def col_slice(c):
            ce = jnp.minimum(c, NCOL - 1) * 128
            return tbl_hbm.at[:, pl.ds(pl.multiple_of(ce, 128), 128)]

        pltpu.async_copy(col_slice(col0), colbuf0, sem0)
        pltpu.async_copy(col_slice(col0 + 1), colbuf1, sem1)

        p_start = _sload(bnd_v, 0)

        def extract(colbuf, gcol, carry):
            hi = _sload(bnd_v, gcol + 1)

            def hit_cond(c):
                return c[0] < hi

            def hit_body(c):
                p, cnt, posvec, nfl = c
                l = _sload(sidx_v, p) & 127
                pos = _sload(perm_v, p)
                row = (nfl & 7) * LANES + cnt
                lvec = jnp.zeros((LANES,), jnp.int32) + l
                for k in range(EMBED_DIM // LANES):
                    v = plsc.load_gather(colbuf, [lanes + k * LANES, lvec])
                    outstage[row, pl.ds(k * LANES, LANES)] = v
                posvec = jnp.where(lanes == cnt, pos, posvec)
                flush = cnt == LANES - 1

                @pl.when(flush)
                def _():
                    posbuf[...] = posvec
                    slot = pl.multiple_of((nfl & 7) * LANES, 8)
                    pltpu.async_copy(outstage.at[pl.ds(slot, LANES)],
                                     emb_hbm.at[posbuf], semsc)

                @pl.when(flush & (nfl >= 7))
                def _():
                    pltpu.make_async_copy(
                        emb_hbm.at[pl.ds(0, LANES)],
                        outstage.at[pl.ds(0, LANES)], semsc).wait()

                posvec = jnp.where(flush, dump_init, posvec)
                cnt = (cnt + 1) & (LANES - 1)
                nfl = nfl + flush.astype(jnp.int32)
                return (p + 1, cnt, posvec, nfl)

            return lax.while_loop(hit_cond, hit_body, carry)

        def col_pair(g, carry):
            c0 = col0 + 2 * g
            pltpu.make_async_copy(col_slice(c0), colbuf0, sem0).wait()
            carry = extract(colbuf0, 2 * g, carry)
            pltpu.async_copy(col_slice(c0 + 2), colbuf0, sem0)
            pltpu.make_async_copy(col_slice(c0 + 1), colbuf1, sem1).wait()
            carry = extract(colbuf1, 2 * g + 1, carry)
            pltpu.async_copy(col_slice(c0 + 3), colbuf1, sem1)
            return carry

        carry = (p_start, jnp.int32(0), dump_init, jnp.int32(0))
        p, cnt, posvec, nfl = lax.fori_loop(
            0, SHARD_COLS // 2, col_pair, carry)

        pltpu.make_async_copy(col_slice(0), colbuf0, sem0).wait()
        pltpu.make_async_copy(col_slice(0), colbuf1, sem1).wait()

        @pl.when(cnt > 0)
        def _():
            posbuf[...] = posvec
            slot = pl.multiple_of((nfl & 7) * LANES, 8)
            pltpu.async_copy(outstage.at[pl.ds(slot, LANES)],
                             emb_hbm.at[posbuf], semsc)

        outstanding = jnp.minimum(nfl, 7) + (cnt > 0).astype(jnp.int32)
        for i in range(8):
            @pl.when(i < outstanding)
            def _():
                pltpu.make_async_copy(
                    emb_hbm.at[pl.ds(0, LANES)],
                    outstage.at[pl.ds(0, LANES)], semsc).wait()

    @pl.when(cid == 0)
    def _():
        process(tin_hbm, sidx_t_hbm, perm_t_hbm, bnd_t_hbm, embt_hbm)

    @pl.when(cid == 1)
    def _():
        process(tout_hbm, sidx_c_hbm, perm_c_hbm, bnd_c_hbm, embc_hbm)


def _dot_body(embt_hbm, embc_hbm, out_hbm, tbuf, cbuf, out_v, sem_t, sem_c):
    wid = lax.axis_index("s") * NUM_CORES + lax.axis_index("c")
    base = wid * BPW
    lanes = lax.iota(jnp.int32, LANES)

    for q in range(BPW // QUARTER):
        q0 = base + q * QUARTER
        cp1 = pltpu.async_copy(embt_hbm.at[pl.ds(q0, QUARTER)], tbuf, sem_t)
        cp2 = pltpu.async_copy(embc_hbm.at[pl.ds(q0, QUARTER)], cbuf, sem_c)
        cp1.wait()
        cp2.wait()

        def block(i, carry):
            r0 = i * LANES
            acc = jnp.zeros((LANES,), jnp.float32)
            for r in range(LANES):
                s = jnp.zeros((LANES,), jnp.float32)
                for k in range(EMBED_DIM // LANES):
                    tv = tbuf[r0 + r, pl.ds(k * LANES, LANES)]
                    cv = cbuf[r0 + r, pl.ds(k * LANES, LANES)]
                    s = s + tv * cv
                tot = jnp.sum(s)
                acc = jnp.where(lanes == r, tot, acc)
            out_v[pl.ds(q * QUARTER + r0, LANES)] = acc
            return carry

        lax.fori_loop(0, QUARTER // LANES, block, 0)

    pltpu.sync_copy(out_v, out_hbm.at[pl.ds(base, BPW)])


@jax.jit
def _word2vec(target, context, tin_t, tout_t):
    mesh = plsc.VectorSubcoreMesh(core_axis_name="c", subcore_axis_name="s")

    def prep(idx):
        perm = jnp.argsort(idx).astype(jnp.int32)
        sidx = jnp.take(idx, perm)
        bnd = jnp.searchsorted(
            sidx, jnp.arange(BOUNDS_PAD, dtype=jnp.int32) * 128,
            side="left").astype(jnp.int32)
        sidx = jnp.concatenate(
            [sidx, jnp.zeros((LANES,), jnp.int32)])
        perm = jnp.concatenate(
            [perm, jnp.zeros((LANES,), jnp.int32)])
        return sidx, perm, bnd

    sidx_t, perm_t, bnd_t = prep(target)
    sidx_c, perm_c, bnd_c = prep(context)

    embt, embc = pl.kernel(
        _gather_body,
        mesh=mesh,
        compiler_params=_COMPILER_PARAMS,
        out_type=(jax.ShapeDtypeStruct((EMB_ROWS, 2 * EMBED_DIM), jnp.float32),
                  jax.ShapeDtypeStruct((EMB_ROWS, 2 * EMBED_DIM), jnp.float32)),
        scratch_types=[
            pltpu.VMEM((BATCH + LANES,), jnp.int32),
            pltpu.VMEM((BATCH + LANES,), jnp.int32),
            pltpu.VMEM((SHARD_COLS + 16,), jnp.int32),
            pltpu.VMEM((EMBED_DIM, 128), jnp.float32),
            pltpu.VMEM((EMBED_DIM, 128), jnp.float32),
            pltpu.VMEM((8 * LANES, 2 * EMBED_DIM), jnp.float32),
            pltpu.VMEM((LANES,), jnp.int32),
            pltpu.SemaphoreType.DMA,
            pltpu.SemaphoreType.DMA,
            pltpu.SemaphoreType.DMA,
        ],
    )(tin_t, tout_t, sidx_t, perm_t, bnd_t, sidx_c, perm_c, bnd_c)

    return pl.kernel(
        _dot_body,
        mesh=mesh,
        compiler_params=_COMPILER_PARAMS,
        out_type=jax.ShapeDtypeStruct((BATCH,), jnp.float32),
        scratch_types=[
            pltpu.VMEM((QUARTER, 2 * EMBED_DIM), jnp.float32),
            pltpu.VMEM((QUARTER, 2 * EMBED_DIM), jnp.float32),
            pltpu.VMEM((BPW,), jnp.float32),
            pltpu.SemaphoreType.DMA,
            pltpu.SemaphoreType.DMA,
        ],
    )(embt, embc)


def kernel(target, context, in_table, out_table):
    return _word2vec(target.astype(jnp.int32), context.astype(jnp.int32),
                     in_table.T, out_table.T)


# speedup vs baseline: 4.1543x; 3.2821x over previous
import functools

import jax
import jax.numpy as jnp
from jax import lax
from jax.experimental import pallas as pl
from jax.experimental.pallas import tpu as pltpu
from jax.experimental.pallas import tpu_sc as plsc

VOCAB = 1000000
EMBED_DIM = 64
BATCH = 16384

NUM_CORES = 2
NUM_SUBCORES = 16
LANES = 16
NUM_WORKERS = NUM_CORES * NUM_SUBCORES
NCOL = (VOCAB + 127) // 128
SHARD_COLS = 496
NBUF = 4
SENTINEL = 0x7FFFFFF0
EMB_ROWS = BATCH + LANES
BPW = BATCH // NUM_WORKERS
QUARTER = 128

_COMPILER_PARAMS = pltpu.CompilerParams(
    needs_layout_passes=False, use_tc_tiling_on_sc=True,
    disable_bounds_checks=True)


def _sload(ref, i):
    return ref[pl.ds(i, LANES)][0]


def _gather_body(tin_hbm, tout_hbm, sidx_t_hbm, perm_t_hbm, bnd_t_hbm,
                 sidx_c_hbm, perm_c_hbm, bnd_c_hbm, embt_hbm, embc_hbm,
                 sidx_v, perm_v, bnd_v, colbufs, outstage, posbuf,
                 semcols, semsc):
    cid = lax.axis_index("c")
    shard = lax.axis_index("s")
    col0 = shard * SHARD_COLS
    lanes = lax.iota(jnp.int32, LANES)
    dump_init = BATCH + lanes

    def process(tbl_hbm, sidx_hbm, perm_hbm, bnd_hbm, emb_hbm):
        pltpu.sync_copy(sidx_hbm, sidx_v)
        pltpu.sync_copy(perm_hbm, perm_v)
        pltpu.sync_copy(bnd_hbm, bnd_v)

        def blk_start(b):
            s = jnp.minimum((col0 + 2 * b) * 128, (NCOL - 2) * 128)
            return pl.multiple_of(s, 128)

        def blk_slice(b):
            return tbl_hbm.at[:, pl.ds(blk_start(b), 2 * 128)]

        for j in range(NBUF):
            pltpu.async_copy(blk_slice(j), colbufs[j], semcols[j])

        p_start = _sload(bnd_v, shard)

        def extract(colbuf, b, carry):
            start = blk_start(b)
            end = (col0 + 2 * b + 2) * 128

            def hit_cond(c):
                return c[4] < end

            def hit_body(c):
                p, cnt, posvec, nfl, cur = c
                l = cur - start
                pos = _sload(perm_v, p)
                row = (nfl & 7) * LANES + cnt
                lvec = jnp.zeros((LANES,), jnp.int32) + l
                for k in range(EMBED_DIM // LANES):
                    v = plsc.load_gather(colbuf, [lanes + k * LANES, lvec])
                    outstage[row, pl.ds(k * LANES, LANES)] = v
                posvec = jnp.where(lanes == cnt, pos, posvec)
                flush = cnt == LANES - 1

                @pl.when(flush)
                def _():
                    posbuf[...] = posvec
                    slot = pl.multiple_of((nfl & 7) * LANES, 8)
                    pltpu.async_copy(outstage.at[pl.ds(slot, LANES)],
                                     emb_hbm.at[posbuf], semsc)

                @pl.when(flush & (nfl >= 7))
                def _():
                    pltpu.make_async_copy(
                        emb_hbm.at[pl.ds(0, LANES)],
                        outstage.at[pl.ds(0, LANES)], semsc).wait()

                posvec = jnp.where(flush, dump_init, posvec)
                cnt = (cnt + 1) & (LANES - 1)
                nfl = nfl + flush.astype(jnp.int32)
                return (p + 1, cnt, posvec, nfl, _sload(sidx_v, p + 1))

            return lax.while_loop(hit_cond, hit_body, carry)

        carry = (p_start, jnp.int32(0), dump_init, jnp.int32(0),
                 _sload(sidx_v, p_start))

        def blk_group(g, carry):
            for j in range(NBUF):
                b = NBUF * g + j
                pltpu.make_async_copy(
                    blk_slice(b), colbufs[j], semcols[j]).wait()
                carry = extract(colbufs[j], b, carry)
                pltpu.async_copy(blk_slice(b + NBUF), colbufs[j], semcols[j])
            return carry

        carry = lax.fori_loop(0, SHARD_COLS // 2 // NBUF, blk_group, carry)
        p, cnt, posvec, nfl, cur = carry

        for j in range(NBUF):
            pltpu.make_async_copy(
                blk_slice(0), colbufs[j], semcols[j]).wait()

        @pl.when(cnt > 0)
        def _():
            posbuf[...] = posvec
            slot = pl.multiple_of((nfl & 7) * LANES, 8)
            pltpu.async_copy(outstage.at[pl.ds(slot, LANES)],
                             emb_hbm.at[posbuf], semsc)

        outstanding = jnp.minimum(nfl, 7) + (cnt > 0).astype(jnp.int32)
        for i in range(8):
            @pl.when(i < outstanding)
            def _():
                pltpu.make_async_copy(
                    emb_hbm.at[pl.ds(0, LANES)],
                    outstage.at[pl.ds(0, LANES)], semsc).wait()

    @pl.when(cid == 0)
    def _():
        process(tin_hbm, sidx_t_hbm, perm_t_hbm, bnd_t_hbm, embt_hbm)

    @pl.when(cid == 1)
    def _():
        process(tout_hbm, sidx_c_hbm, perm_c_hbm, bnd_c_hbm, embc_hbm)


def _dot_body(embt_hbm, embc_hbm, out_hbm, tbuf, cbuf, out_v, sem_t, sem_c):
    wid = lax.axis_index("s") * NUM_CORES + lax.axis_index("c")
    base = wid * BPW
    lanes = lax.iota(jnp.int32, LANES)

    for q in range(BPW // QUARTER):
        q0 = base + q * QUARTER
        cp1 = pltpu.async_copy(embt_hbm.at[pl.ds(q0, QUARTER)], tbuf, sem_t)
        cp2 = pltpu.async_copy(embc_hbm.at[pl.ds(q0, QUARTER)], cbuf, sem_c)
        cp1.wait()
        cp2.wait()

        def block(i, carry):
            r0 = i * LANES
            acc = jnp.zeros((LANES,), jnp.float32)
            for r in range(LANES):
                s = jnp.zeros((LANES,), jnp.float32)
                for k in range(EMBED_DIM // LANES):
                    tv = tbuf[r0 + r, pl.ds(k * LANES, LANES)]
                    cv = cbuf[r0 + r, pl.ds(k * LANES, LANES)]
                    s = s + tv * cv
                tot = jnp.sum(s)
                acc = jnp.where(lanes == r, tot, acc)
            out_v[pl.ds(q * QUARTER + r0, LANES)] = acc
            return carry

        lax.fori_loop(0, QUARTER // LANES, block, 0)

    pltpu.sync_copy(out_v, out_hbm.at[pl.ds(base, BPW)])


@jax.jit
def _word2vec(target, context, tin_t, tout_t):
    mesh = plsc.VectorSubcoreMesh(core_axis_name="c", subcore_axis_name="s")

    def prep(idx):
        sidx, perm = lax.sort(
            (idx, jnp.arange(BATCH, dtype=jnp.int32)), num_keys=1)
        bnd = jnp.searchsorted(
            sidx, jnp.arange(32, dtype=jnp.int32) * (SHARD_COLS * 128),
            side="left").astype(jnp.int32)
        sidx = jnp.concatenate(
            [sidx, jnp.full((LANES,), SENTINEL, jnp.int32)])
        perm = jnp.concatenate(
            [perm, jnp.zeros((LANES,), jnp.int32)])
        return sidx, perm, bnd

    sidx_t, perm_t, bnd_t = prep(target)
    sidx_c, perm_c, bnd_c = prep(context)

    embt, embc = pl.kernel(
        _gather_body,
        mesh=mesh,
        compiler_params=_COMPILER_PARAMS,
        out_type=(jax.ShapeDtypeStruct((EMB_ROWS, 2 * EMBED_DIM), jnp.float32),
                  jax.ShapeDtypeStruct((EMB_ROWS, 2 * EMBED_DIM), jnp.float32)),
        scratch_types=[
            pltpu.VMEM((BATCH + LANES,), jnp.int32),
            pltpu.VMEM((BATCH + LANES,), jnp.int32),
            pltpu.VMEM((32,), jnp.int32),
            [pltpu.VMEM((EMBED_DIM, 2 * 128), jnp.float32)
             for _ in range(NBUF)],
            pltpu.VMEM((8 * LANES, 2 * EMBED_DIM), jnp.float32),
            pltpu.VMEM((LANES,), jnp.int32),
            [pltpu.SemaphoreType.DMA for _ in range(NBUF)],
            pltpu.SemaphoreType.DMA,
        ],
    )(tin_t, tout_t, sidx_t, perm_t, bnd_t, sidx_c, perm_c, bnd_c)

    return pl.kernel(
        _dot_body,
        mesh=mesh,
        compiler_params=_COMPILER_PARAMS,
        out_type=jax.ShapeDtypeStruct((BATCH,), jnp.float32),
        scratch_types=[
            pltpu.VMEM((QUARTER, 2 * EMBED_DIM), jnp.float32),
            pltpu.VMEM((QUARTER, 2 * EMBED_DIM), jnp.float32),
            pltpu.VMEM((BPW,), jnp.float32),
            pltpu.SemaphoreType.DMA,
            pltpu.SemaphoreType.DMA,
        ],
    )(embt, embc)


def kernel(target, context, in_table, out_table):
    return _word2vec(target.astype(jnp.int32), context.astype(jnp.int32),
                     in_table.T, out_table.T)


# speedup vs baseline: 4.2410x; 1.0209x over previous
import functools

import jax
import jax.numpy as jnp
from jax import lax
from jax.experimental import pallas as pl
from jax.experimental.pallas import tpu as pltpu
from jax.experimental.pallas import tpu_sc as plsc

VOCAB = 1000000
EMBED_DIM = 64
BATCH = 16384

NUM_CORES = 2
NUM_SUBCORES = 16
LANES = 16
NUM_WORKERS = NUM_CORES * NUM_SUBCORES
NCOL = (VOCAB + 127) // 128
SHARD_COLS = 500
NBUF = 5
NSLOT = 4
SENTINEL = 0x7FFFFFF0
EMB_ROWS = BATCH + LANES
BPW = BATCH // NUM_WORKERS
QUARTER = 128

_COMPILER_PARAMS = pltpu.CompilerParams(
    needs_layout_passes=False, use_tc_tiling_on_sc=True,
    disable_bounds_checks=True)


def _sload(ref, i):
    return ref[pl.ds(i, LANES)][0]


def _gather_body(tin_hbm, tout_hbm, sidx_t_hbm, perm_t_hbm, bnd_t_hbm,
                 sidx_c_hbm, perm_c_hbm, bnd_c_hbm, embt_hbm, embc_hbm,
                 sidx_v, perm_v, bnd_v, colbufs, outstage, posbuf,
                 semcols, semsc):
    cid = lax.axis_index("c")
    shard = lax.axis_index("s")
    col0 = shard * SHARD_COLS
    lanes = lax.iota(jnp.int32, LANES)
    dump_init = BATCH + lanes

    def process(tbl_hbm, sidx_hbm, perm_hbm, bnd_hbm, emb_hbm):
        pltpu.sync_copy(sidx_hbm, sidx_v)
        pltpu.sync_copy(perm_hbm, perm_v)
        pltpu.sync_copy(bnd_hbm, bnd_v)

        def blk_start(b):
            s = jnp.minimum((col0 + 2 * b) * 128, (NCOL - 2) * 128)
            return pl.multiple_of(s, 128)

        def blk_slice(b):
            return tbl_hbm.at[:, pl.ds(blk_start(b), 2 * 128)]

        for j in range(NBUF):
            pltpu.async_copy(blk_slice(j), colbufs[j], semcols[j])

        p_start = _sload(bnd_v, shard)

        def extract(colbuf, b, carry):
            start = blk_start(b)
            end = (col0 + 2 * b + 2) * 128

            def hit_cond(c):
                return c[4] < end

            def hit_body(c):
                p, cnt, posvec, nfl, cur = c
                l = cur - start
                pos = _sload(perm_v, p)
                row = (nfl & (NSLOT - 1)) * LANES + cnt
                lvec = jnp.zeros((LANES,), jnp.int32) + l
                for k in range(EMBED_DIM // LANES):
                    v = plsc.load_gather(colbuf, [lanes + k * LANES, lvec])
                    outstage[row, pl.ds(k * LANES, LANES)] = v
                posvec = jnp.where(lanes == cnt, pos, posvec)
                flush = cnt == LANES - 1

                @pl.when(flush)
                def _():
                    posbuf[...] = posvec
                    slot = pl.multiple_of((nfl & (NSLOT - 1)) * LANES, 8)
                    pltpu.async_copy(outstage.at[pl.ds(slot, LANES)],
                                     emb_hbm.at[posbuf], semsc)

                @pl.when(flush & (nfl >= NSLOT - 1))
                def _():
                    pltpu.make_async_copy(
                        emb_hbm.at[pl.ds(0, LANES)],
                        outstage.at[pl.ds(0, LANES)], semsc).wait()

                posvec = jnp.where(flush, dump_init, posvec)
                cnt = (cnt + 1) & (LANES - 1)
                nfl = nfl + flush.astype(jnp.int32)
                return (p + 1, cnt, posvec, nfl, _sload(sidx_v, p + 1))

            return lax.while_loop(hit_cond, hit_body, carry)

        carry = (p_start, jnp.int32(0), dump_init, jnp.int32(0),
                 _sload(sidx_v, p_start))

        def blk_group(g, carry):
            for j in range(NBUF):
                b = NBUF * g + j
                pltpu.make_async_copy(
                    blk_slice(b), colbufs[j], semcols[j]).wait()
                carry = extract(colbufs[j], b, carry)
                pltpu.async_copy(blk_slice(b + NBUF), colbufs[j], semcols[j])
            return carry

        carry = lax.fori_loop(0, SHARD_COLS // 2 // NBUF, blk_group, carry)
        p, cnt, posvec, nfl, cur = carry

        for j in range(NBUF):
            pltpu.make_async_copy(
                blk_slice(0), colbufs[j], semcols[j]).wait()

        @pl.when(cnt > 0)
        def _():
            posbuf[...] = posvec
            slot = pl.multiple_of((nfl & (NSLOT - 1)) * LANES, 8)
            pltpu.async_copy(outstage.at[pl.ds(slot, LANES)],
                             emb_hbm.at[posbuf], semsc)

        outstanding = (jnp.minimum(nfl, NSLOT - 1)
                       + (cnt > 0).astype(jnp.int32))
        for i in range(NSLOT):
            @pl.when(i < outstanding)
            def _():
                pltpu.make_async_copy(
                    emb_hbm.at[pl.ds(0, LANES)],
                    outstage.at[pl.ds(0, LANES)], semsc).wait()

    @pl.when(cid == 0)
    def _():
        process(tin_hbm, sidx_t_hbm, perm_t_hbm, bnd_t_hbm, embt_hbm)

    @pl.when(cid == 1)
    def _():
        process(tout_hbm, sidx_c_hbm, perm_c_hbm, bnd_c_hbm, embc_hbm)


def _dot_body(embt_hbm, embc_hbm, out_hbm, tbuf, cbuf, out_v, sem_t, sem_c):
    wid = lax.axis_index("s") * NUM_CORES + lax.axis_index("c")
    base = wid * BPW
    lanes = lax.iota(jnp.int32, LANES)

    for q in range(BPW // QUARTER):
        q0 = base + q * QUARTER
        cp1 = pltpu.async_copy(embt_hbm.at[pl.ds(q0, QUARTER)], tbuf, sem_t)
        cp2 = pltpu.async_copy(embc_hbm.at[pl.ds(q0, QUARTER)], cbuf, sem_c)
        cp1.wait()
        cp2.wait()

        def block(i, carry):
            r0 = i * LANES
            acc = jnp.zeros((LANES,), jnp.float32)
            for r in range(LANES):
                s = jnp.zeros((LANES,), jnp.float32)
                for k in range(EMBED_DIM // LANES):
                    tv = tbuf[r0 + r, pl.ds(k * LANES, LANES)]
                    cv = cbuf[r0 + r, pl.ds(k * LANES, LANES)]
                    s = s + tv * cv
                tot = jnp.sum(s)
                acc = jnp.where(lanes == r, tot, acc)
            out_v[pl.ds(q * QUARTER + r0, LANES)] = acc
            return carry

        lax.fori_loop(0, QUARTER // LANES, block, 0)

    pltpu.sync_copy(out_v, out_hbm.at[pl.ds(base, BPW)])


@jax.jit
def _word2vec(target, context, tin_t, tout_t):
    mesh = plsc.VectorSubcoreMesh(core_axis_name="c", subcore_axis_name="s")

    def prep(idx):
        sidx, perm = lax.sort(
            (idx, jnp.arange(BATCH, dtype=jnp.int32)), num_keys=1)
        bnd = jnp.searchsorted(
            sidx, jnp.arange(32, dtype=jnp.int32) * (SHARD_COLS * 128),
            side="left").astype(jnp.int32)
        sidx = jnp.concatenate(
            [sidx, jnp.full((LANES,), SENTINEL, jnp.int32)])
        perm = jnp.concatenate(
            [perm, jnp.zeros((LANES,), jnp.int32)])
        return sidx, perm, bnd

    sidx_t, perm_t, bnd_t = prep(target)
    sidx_c, perm_c, bnd_c = prep(context)

    embt, embc = pl.kernel(
        _gather_body,
        mesh=mesh,
        compiler_params=_COMPILER_PARAMS,
        out_type=(jax.ShapeDtypeStruct((EMB_ROWS, 2 * EMBED_DIM), jnp.float32),
                  jax.ShapeDtypeStruct((EMB_ROWS, 2 * EMBED_DIM), jnp.float32)),
        scratch_types=[
            pltpu.VMEM((BATCH + LANES,), jnp.int32),
            pltpu.VMEM((BATCH + LANES,), jnp.int32),
            pltpu.VMEM((32,), jnp.int32),
            [pltpu.VMEM((EMBED_DIM, 2 * 128), jnp.float32)
             for _ in range(NBUF)],
            pltpu.VMEM((NSLOT * LANES, 2 * EMBED_DIM), jnp.float32),
            pltpu.VMEM((LANES,), jnp.int32),
            [pltpu.SemaphoreType.DMA for _ in range(NBUF)],
            pltpu.SemaphoreType.DMA,
        ],
    )(tin_t, tout_t, sidx_t, perm_t, bnd_t, sidx_c, perm_c, bnd_c)

    return pl.kernel(
        _dot_body,
        mesh=mesh,
        compiler_params=_COMPILER_PARAMS,
        out_type=jax.ShapeDtypeStruct((BATCH,), jnp.float32),
        scratch_types=[
            pltpu.VMEM((QUARTER, 2 * EMBED_DIM), jnp.float32),
            pltpu.VMEM((QUARTER, 2 * EMBED_DIM), jnp.float32),
            pltpu.VMEM((BPW,), jnp.float32),
            pltpu.SemaphoreType.DMA,
            pltpu.SemaphoreType.DMA,
        ],
    )(embt, embc)


def kernel(target, context, in_table, out_table):
    return _word2vec(target.astype(jnp.int32), context.astype(jnp.int32),
                     in_table.T, out_table.T)


# speedup vs baseline: 4.2886x; 1.0112x over previous
import functools

import jax
import jax.numpy as jnp
from jax import lax
from jax.experimental import pallas as pl
from jax.experimental.pallas import tpu as pltpu
from jax.experimental.pallas import tpu_sc as plsc

VOCAB = 1000000
EMBED_DIM = 64
BATCH = 16384

NUM_CORES = 2
NUM_SUBCORES = 16
LANES = 16
NUM_WORKERS = NUM_CORES * NUM_SUBCORES
NCOL = (VOCAB + 127) // 128
SHARD_COLS = 500
NBUF = 5
NSLOT = 4
KEY_POS = 16384
SENTINEL = 0x7FFFFFF0
EMB_ROWS = BATCH + LANES
BPW = BATCH // NUM_WORKERS
QUARTER = 128

_COMPILER_PARAMS = pltpu.CompilerParams(
    needs_layout_passes=False, use_tc_tiling_on_sc=True,
    disable_bounds_checks=True)


def _sload(ref, i):
    return ref[pl.ds(i, LANES)][0]


def _gather_body(tin_hbm, tout_hbm, sidx_t_hbm, perm_t_hbm, bnd_t_hbm,
                 sidx_c_hbm, perm_c_hbm, bnd_c_hbm, embt_hbm, embc_hbm,
                 sidx_v, perm_v, bnd_v, colbufs, outstage, posbuf,
                 semcols, semsc):
    cid = lax.axis_index("c")
    shard = lax.axis_index("s")
    col0 = shard * SHARD_COLS
    lanes = lax.iota(jnp.int32, LANES)
    dump_init = BATCH + lanes

    def process(tbl_hbm, sidx_hbm, perm_hbm, bnd_hbm, emb_hbm):
        pltpu.sync_copy(sidx_hbm, sidx_v)
        pltpu.sync_copy(perm_hbm, perm_v)
        pltpu.sync_copy(bnd_hbm, bnd_v)

        def blk_start(b):
            s = jnp.minimum((col0 + 2 * b) * 128, (NCOL - 2) * 128)
            return pl.multiple_of(s, 128)

        def blk_slice(b):
            return tbl_hbm.at[:, pl.ds(blk_start(b), 2 * 128)]

        for j in range(NBUF):
            pltpu.async_copy(blk_slice(j), colbufs[j], semcols[j])

        p_start = _sload(bnd_v, shard)

        def extract(colbuf, b, carry):
            start = blk_start(b)
            end = (col0 + 2 * b + 2) * KEY_POS

            def hit_cond(c):
                return c[4] < end

            def hit_body(c):
                p, cnt, posvec, nfl, cur = c
                pos = cur & (KEY_POS - 1)
                l = _sload(perm_v, pos) - start
                row = (nfl & (NSLOT - 1)) * LANES + cnt
                lvec = jnp.zeros((LANES,), jnp.int32) + l
                for k in range(EMBED_DIM // LANES):
                    v = plsc.load_gather(colbuf, [lanes + k * LANES, lvec])
                    outstage[row, pl.ds(k * LANES, LANES)] = v
                posvec = jnp.where(lanes == cnt, pos, posvec)
                flush = cnt == LANES - 1

                @pl.when(flush)
                def _():
                    posbuf[...] = posvec
                    slot = pl.multiple_of((nfl & (NSLOT - 1)) * LANES, 8)
                    pltpu.async_copy(outstage.at[pl.ds(slot, LANES)],
                                     emb_hbm.at[posbuf], semsc)

                @pl.when(flush & (nfl >= NSLOT - 1))
                def _():
                    pltpu.make_async_copy(
                        emb_hbm.at[pl.ds(0, LANES)],
                        outstage.at[pl.ds(0, LANES)], semsc).wait()

                posvec = jnp.where(flush, dump_init, posvec)
                cnt = (cnt + 1) & (LANES - 1)
                nfl = nfl + flush.astype(jnp.int32)
                return (p + 1, cnt, posvec, nfl, _sload(sidx_v, p + 1))

            return lax.while_loop(hit_cond, hit_body, carry)

        carry = (p_start, jnp.int32(0), dump_init, jnp.int32(0),
                 _sload(sidx_v, p_start))

        def blk_group(g, carry):
            for j in range(NBUF):
                b = NBUF * g + j
                pltpu.make_async_copy(
                    blk_slice(b), colbufs[j], semcols[j]).wait()
                carry = extract(colbufs[j], b, carry)
                pltpu.async_copy(blk_slice(b + NBUF), colbufs[j], semcols[j])
            return carry

        carry = lax.fori_loop(0, SHARD_COLS // 2 // NBUF, blk_group, carry)
        p, cnt, posvec, nfl, cur = carry

        for j in range(NBUF):
            pltpu.make_async_copy(
                blk_slice(0), colbufs[j], semcols[j]).wait()

        @pl.when(cnt > 0)
        def _():
            posbuf[...] = posvec
            slot = pl.multiple_of((nfl & (NSLOT - 1)) * LANES, 8)
            pltpu.async_copy(outstage.at[pl.ds(slot, LANES)],
                             emb_hbm.at[posbuf], semsc)

        outstanding = (jnp.minimum(nfl, NSLOT - 1)
                       + (cnt > 0).astype(jnp.int32))
        for i in range(NSLOT):
            @pl.when(i < outstanding)
            def _():
                pltpu.make_async_copy(
                    emb_hbm.at[pl.ds(0, LANES)],
                    outstage.at[pl.ds(0, LANES)], semsc).wait()

    @pl.when(cid == 0)
    def _():
        process(tin_hbm, sidx_t_hbm, perm_t_hbm, bnd_t_hbm, embt_hbm)

    @pl.when(cid == 1)
    def _():
        process(tout_hbm, sidx_c_hbm, perm_c_hbm, bnd_c_hbm, embc_hbm)


def _dot_body(embt_hbm, embc_hbm, out_hbm, tbufs, cbufs, out_v, sem_t, sem_c):
    wid = lax.axis_index("s") * NUM_CORES + lax.axis_index("c")
    base = wid * BPW
    lanes = lax.iota(jnp.int32, LANES)

    nq = BPW // QUARTER

    def fire(q):
        q0 = base + q * QUARTER
        return (pltpu.async_copy(embt_hbm.at[pl.ds(q0, QUARTER)],
                                 tbufs[q % 2], sem_t),
                pltpu.async_copy(embc_hbm.at[pl.ds(q0, QUARTER)],
                                 cbufs[q % 2], sem_c))

    pending = fire(0)
    for q in range(nq):
        tbuf, cbuf = tbufs[q % 2], cbufs[q % 2]
        cp1, cp2 = pending
        cp1.wait()
        cp2.wait()
        if q + 1 < nq:
            pending = fire(q + 1)

        def block(i, carry):
            r0 = i * LANES
            acc = jnp.zeros((LANES,), jnp.float32)
            for r in range(LANES):
                s = jnp.zeros((LANES,), jnp.float32)
                for k in range(EMBED_DIM // LANES):
                    tv = tbuf[r0 + r, pl.ds(k * LANES, LANES)]
                    cv = cbuf[r0 + r, pl.ds(k * LANES, LANES)]
                    s = s + tv * cv
                tot = jnp.sum(s)
                acc = jnp.where(lanes == r, tot, acc)
            out_v[pl.ds(q * QUARTER + r0, LANES)] = acc
            return carry

        lax.fori_loop(0, QUARTER // LANES, block, 0)

    pltpu.sync_copy(out_v, out_hbm.at[pl.ds(base, BPW)])


@jax.jit
def _word2vec(target, context, tin_t, tout_t):
    mesh = plsc.VectorSubcoreMesh(core_axis_name="c", subcore_axis_name="s")

    def prep(idx):
        keys = ((idx >> 7) * KEY_POS
                + jnp.arange(BATCH, dtype=jnp.int32))
        skeys = lax.sort(keys)
        bnd = jnp.searchsorted(
            skeys,
            jnp.arange(32, dtype=jnp.int32) * (SHARD_COLS * KEY_POS),
            side="left").astype(jnp.int32)
        skeys = jnp.concatenate(
            [skeys, jnp.full((LANES,), SENTINEL, jnp.int32)])
        idx_pad = jnp.concatenate(
            [idx, jnp.zeros((LANES,), jnp.int32)])
        return skeys, idx_pad, bnd

    sidx_t, perm_t, bnd_t = prep(target)
    sidx_c, perm_c, bnd_c = prep(context)

    embt, embc = pl.kernel(
        _gather_body,
        mesh=mesh,
        compiler_params=_COMPILER_PARAMS,
        out_type=(jax.ShapeDtypeStruct((EMB_ROWS, 2 * EMBED_DIM), jnp.float32),
                  jax.ShapeDtypeStruct((EMB_ROWS, 2 * EMBED_DIM), jnp.float32)),
        scratch_types=[
            pltpu.VMEM((BATCH + LANES,), jnp.int32),
            pltpu.VMEM((BATCH + LANES,), jnp.int32),
            pltpu.VMEM((32,), jnp.int32),
            [pltpu.VMEM((EMBED_DIM, 2 * 128), jnp.float32)
             for _ in range(NBUF)],
            pltpu.VMEM((NSLOT * LANES, 2 * EMBED_DIM), jnp.float32),
            pltpu.VMEM((LANES,), jnp.int32),
            [pltpu.SemaphoreType.DMA for _ in range(NBUF)],
            pltpu.SemaphoreType.DMA,
        ],
    )(tin_t, tout_t, sidx_t, perm_t, bnd_t, sidx_c, perm_c, bnd_c)

    return pl.kernel(
        _dot_body,
        mesh=mesh,
        compiler_params=_COMPILER_PARAMS,
        out_type=jax.ShapeDtypeStruct((BATCH,), jnp.float32),
        scratch_types=[
            [pltpu.VMEM((QUARTER, 2 * EMBED_DIM), jnp.float32)
             for _ in range(2)],
            [pltpu.VMEM((QUARTER, 2 * EMBED_DIM), jnp.float32)
             for _ in range(2)],
            pltpu.VMEM((BPW,), jnp.float32),
            pltpu.SemaphoreType.DMA,
            pltpu.SemaphoreType.DMA,
        ],
    )(embt, embc)


def kernel(target, context, in_table, out_table):
    return _word2vec(target.astype(jnp.int32), context.astype(jnp.int32),
                     in_table.T, out_table.T)


# speedup vs baseline: 4.3014x; 1.0030x over previous
import jax
import jax.numpy as jnp
from jax import lax
from jax.experimental import pallas as pl
from jax.experimental.pallas import tpu as pltpu
from jax.experimental.pallas import tpu_sc as plsc

VOCAB = 1000000
EMBED_DIM = 64
BATCH = 16384

NUM_CORES = 2
NUM_SUBCORES = 16
LANES = 16
NUM_WORKERS = NUM_CORES * NUM_SUBCORES
NCOL = (VOCAB + 127) // 128
SHARD_COLS = 500
NBUF = 5
NSLOT = 4
KEY_POS = 16384
SENTINEL = 0x7FFFFFF0
EMB_ROWS = BATCH + LANES
BPW = BATCH // NUM_WORKERS
QUARTER = 128

_COMPILER_PARAMS = pltpu.CompilerParams(
    needs_layout_passes=False, use_tc_tiling_on_sc=True,
    disable_bounds_checks=True)


def _sload(ref, i):
    return ref[pl.ds(i, LANES)][0]


def _gather_body(tin_hbm, tout_hbm, sidx_t_hbm, perm_t_hbm, bnd_t_hbm,
                 sidx_c_hbm, perm_c_hbm, bnd_c_hbm, embt_hbm, embc_hbm,
                 sidx_v, perm_v, bnd_v, colbufs, outstage, posbuf,
                 semcols, semsc):
    cid = lax.axis_index("c")
    shard = lax.axis_index("s")
    col0 = shard * SHARD_COLS
    lanes = lax.iota(jnp.int32, LANES)
    dump_init = BATCH + lanes

    def process(tbl_hbm, sidx_hbm, perm_hbm, bnd_hbm, emb_hbm):
        pltpu.sync_copy(sidx_hbm, sidx_v)
        pltpu.sync_copy(perm_hbm, perm_v)
        pltpu.sync_copy(bnd_hbm, bnd_v)

        def blk_start(b):
            s = jnp.minimum((col0 + 2 * b) * 128, (NCOL - 2) * 128)
            return pl.multiple_of(s, 128)

        def blk_slice(b):
            return tbl_hbm.at[:, pl.ds(blk_start(b), 2 * 128)]

        for j in range(NBUF):
            pltpu.async_copy(blk_slice(j), colbufs[j], semcols[j])

        p_start = _sload(bnd_v, shard)

        def extract(colbuf, b, carry):
            start = blk_start(b)
            end = (col0 + 2 * b + 2) * KEY_POS

            def hit_cond(c):
                return c[4] < end

            def hit_body(c):
                p, cnt, posvec, nfl, cur = c
                pos = cur & (KEY_POS - 1)
                l = _sload(perm_v, pos) - start
                row = (nfl & (NSLOT - 1)) * LANES + cnt
                lvec = jnp.zeros((LANES,), jnp.int32) + l
                for k in range(EMBED_DIM // LANES):
                    v = plsc.load_gather(colbuf, [lanes + k * LANES, lvec])
                    outstage[row, pl.ds(k * LANES, LANES)] = v
                posvec = jnp.where(lanes == cnt, pos, posvec)
                flush = cnt == LANES - 1

                @pl.when(flush)
                def _():
                    posbuf[...] = posvec
                    slot = pl.multiple_of((nfl & (NSLOT - 1)) * LANES, 8)
                    pltpu.async_copy(outstage.at[pl.ds(slot, LANES)],
                                     emb_hbm.at[posbuf], semsc)

                @pl.when(flush & (nfl >= NSLOT - 1))
                def _():
                    pltpu.make_async_copy(
                        emb_hbm.at[pl.ds(0, LANES)],
                        outstage.at[pl.ds(0, LANES)], semsc).wait()

                posvec = jnp.where(flush, dump_init, posvec)
                cnt = (cnt + 1) & (LANES - 1)
                nfl = nfl + flush.astype(jnp.int32)
                return (p + 1, cnt, posvec, nfl, _sload(sidx_v, p + 1))

            return lax.while_loop(hit_cond, hit_body, carry)

        carry = (p_start, jnp.int32(0), dump_init, jnp.int32(0),
                 _sload(sidx_v, p_start))

        def blk_group(g, carry):
            for j in range(NBUF):
                b = NBUF * g + j
                pltpu.make_async_copy(
                    blk_slice(b), colbufs[j], semcols[j]).wait()
                carry = extract(colbufs[j], b, carry)
                pltpu.async_copy(blk_slice(b + NBUF), colbufs[j], semcols[j])
            return carry

        carry = lax.fori_loop(0, SHARD_COLS // 2 // NBUF, blk_group, carry)
        p, cnt, posvec, nfl, cur = carry

        for j in range(NBUF):
            pltpu.make_async_copy(
                blk_slice(0), colbufs[j], semcols[j]).wait()

        @pl.when(cnt > 0)
        def _():
            posbuf[...] = posvec
            slot = pl.multiple_of((nfl & (NSLOT - 1)) * LANES, 8)
            pltpu.async_copy(outstage.at[pl.ds(slot, LANES)],
                             emb_hbm.at[posbuf], semsc)

        outstanding = (jnp.minimum(nfl, NSLOT - 1)
                       + (cnt > 0).astype(jnp.int32))
        for i in range(NSLOT):
            @pl.when(i < outstanding)
            def _():
                pltpu.make_async_copy(
                    emb_hbm.at[pl.ds(0, LANES)],
                    outstage.at[pl.ds(0, LANES)], semsc).wait()

    @pl.when(cid == 0)
    def _():
        process(tin_hbm, sidx_t_hbm, perm_t_hbm, bnd_t_hbm, embt_hbm)

    @pl.when(cid == 1)
    def _():
        process(tout_hbm, sidx_c_hbm, perm_c_hbm, bnd_c_hbm, embc_hbm)


def _dot_body(embt_hbm, embc_hbm, out_hbm, tbufs, cbufs, out_v, sem_t, sem_c):
    wid = lax.axis_index("s") * NUM_CORES + lax.axis_index("c")
    base = wid * BPW
    lanes = lax.iota(jnp.int32, LANES)

    nq = BPW // QUARTER

    def fire(q):
        q0 = base + q * QUARTER
        return (pltpu.async_copy(embt_hbm.at[pl.ds(q0, QUARTER)],
                                 tbufs[q % 2], sem_t),
                pltpu.async_copy(embc_hbm.at[pl.ds(q0, QUARTER)],
                                 cbufs[q % 2], sem_c))

    pending = fire(0)
    for q in range(nq):
        tbuf, cbuf = tbufs[q % 2], cbufs[q % 2]
        cp1, cp2 = pending
        cp1.wait()
        cp2.wait()
        if q + 1 < nq:
            pending = fire(q + 1)

        def block(i, carry):
            r0 = i * LANES
            acc = jnp.zeros((LANES,), jnp.float32)
            for r in range(LANES):
                s = jnp.zeros((LANES,), jnp.float32)
                for k in range(EMBED_DIM // LANES):
                    tv = tbuf[r0 + r, pl.ds(k * LANES, LANES)]
                    cv = cbuf[r0 + r, pl.ds(k * LANES, LANES)]
                    s = s + tv * cv
                tot = jnp.sum(s)
                acc = jnp.where(lanes == r, tot, acc)
            out_v[pl.ds(q * QUARTER + r0, LANES)] = acc
            return carry

        lax.fori_loop(0, QUARTER // LANES, block, 0)

    pltpu.sync_copy(out_v, out_hbm.at[pl.ds(base, BPW)])


@jax.jit
def _word2vec(target, context, tin_t, tout_t):
    mesh = plsc.VectorSubcoreMesh(core_axis_name="c", subcore_axis_name="s")

    def prep(idx):
        keys = ((idx >> 7) * KEY_POS
                + jnp.arange(BATCH, dtype=jnp.int32))
        skeys = lax.sort(keys)
        bnd = jnp.searchsorted(
            skeys,
            jnp.arange(32, dtype=jnp.int32) * (SHARD_COLS * KEY_POS),
            side="left").astype(jnp.int32)
        skeys = jnp.concatenate(
            [skeys, jnp.full((LANES,), SENTINEL, jnp.int32)])
        idx_pad = jnp.concatenate(
            [idx, jnp.zeros((LANES,), jnp.int32)])
        return skeys, idx_pad, bnd

    sidx_t, perm_t, bnd_t = prep(target)
    sidx_c, perm_c, bnd_c = prep(context)

    embt, embc = pl.kernel(
        _gather_body,
        mesh=mesh,
        compiler_params=_COMPILER_PARAMS,
        out_type=(jax.ShapeDtypeStruct((EMB_ROWS, 2 * EMBED_DIM), jnp.float32),
                  jax.ShapeDtypeStruct((EMB_ROWS, 2 * EMBED_DIM), jnp.float32)),
        scratch_types=[
            pltpu.VMEM((BATCH + LANES,), jnp.int32),
            pltpu.VMEM((BATCH + LANES,), jnp.int32),
            pltpu.VMEM((32,), jnp.int32),
            [pltpu.VMEM((EMBED_DIM, 2 * 128), jnp.float32)
             for _ in range(NBUF)],
            pltpu.VMEM((NSLOT * LANES, 2 * EMBED_DIM), jnp.float32),
            pltpu.VMEM((LANES,), jnp.int32),
            [pltpu.SemaphoreType.DMA for _ in range(NBUF)],
            pltpu.SemaphoreType.DMA,
        ],
    )(tin_t, tout_t, sidx_t, perm_t, bnd_t, sidx_c, perm_c, bnd_c)

    return pl.kernel(
        _dot_body,
        mesh=mesh,
        compiler_params=_COMPILER_PARAMS,
        out_type=jax.ShapeDtypeStruct((BATCH,), jnp.float32),
        scratch_types=[
            [pltpu.VMEM((QUARTER, 2 * EMBED_DIM), jnp.float32)
             for _ in range(2)],
            [pltpu.VMEM((QUARTER, 2 * EMBED_DIM), jnp.float32)
             for _ in range(2)],
            pltpu.VMEM((BPW,), jnp.float32),
            pltpu.SemaphoreType.DMA,
            pltpu.SemaphoreType.DMA,
        ],
    )(embt, embc)


def kernel(target, context, in_table, out_table):
    return _word2vec(target.astype(jnp.int32), context.astype(jnp.int32),
                     in_table.T, out_table.T)
